# baseline (speedup 1.0000x reference)
import jax
import jax.numpy as jnp
import numpy as np
from jax import lax
from jax.experimental import pallas as pl
from jax.experimental.pallas import tpu as pltpu
from jax.experimental.pallas import tpu_sc as plsc

_PRIME = 1000000007
_VOCAB = 32768
_DIM = 512
_BATCH = 4
_SEQ = 8192
_NW = 32
_BPW = (_BATCH * _SEQ) // _NW
_CHUNK = 32
_NCHUNK = _BPW // _CHUNK
_ROWPAD = _SEQ + 16
_WPR = _SEQ // _BPW

_POW = [pow(_PRIME, j, 2 ** 32) for j in range(5)]
_P = [np.int32(x - 2 ** 32 if x >= 2 ** 31 else x) for x in _POW]


def _body(tok_hbm, w0, w1, w2, out_hbm, tok_v, i0_v, i1_v, i2_v,
          r0a, r1a, r2a, r0b, r1b, r2b, oa, ob, gsa, gsb, wsa, wsb):
    c32 = jnp.int32
    wid = lax.axis_index("s") * c32(2) + lax.axis_index("c")
    brow = wid // c32(_WPR)
    coff = wid % c32(_WPR)
    tok_base = brow * c32(_ROWPAD) + coff * c32(_BPW)
    pltpu.sync_copy(tok_hbm.at[pl.ds(tok_base, _BPW + 8)], tok_v)

    def hash_step(i, _):
        o = i * jnp.int32(16)
        t0 = tok_v[pl.ds(o + 4, 16)]
        t1 = tok_v[pl.ds(o + 3, 16)]
        t2 = tok_v[pl.ds(o + 2, 16)]
        t3 = tok_v[pl.ds(o + 1, 16)]
        t4 = tok_v[pl.ds(o, 16)]
        h3 = t0 * _P[2] + t1 * _P[1] + t2
        h4 = t0 * _P[3] + t1 * _P[2] + t2 * _P[1] + t3
        h5 = t0 * _P[4] + t1 * _P[3] + t2 * _P[2] + t3 * _P[1] + t4
        m = jnp.int32(0x7FFF)
        i0_v[pl.ds(o, 16)] = jnp.bitwise_and(h3, m)
        i1_v[pl.ds(o, 16)] = jnp.bitwise_and(h4, m)
        i2_v[pl.ds(o, 16)] = jnp.bitwise_and(h5, m)
        return jnp.int32(0)

    lax.fori_loop(jnp.int32(0), jnp.int32(_BPW // 16), hash_step,
                  jnp.int32(0))

    out_base = wid * jnp.int32(_BPW)
    R0 = [r0a, r0b]
    R1 = [r1a, r1b]
    R2 = [r2a, r2b]
    GS = [gsa, gsb]
    WS = [wsa, wsb]

    def issue_gathers(o, s):
        pltpu.async_copy(w0.at[i0_v.at[pl.ds(o, _CHUNK)]], R0[s], GS[s])
        pltpu.async_copy(w1.at[i1_v.at[pl.ds(o, _CHUNK)]], R1[s], GS[s])
        pltpu.async_copy(w2.at[i2_v.at[pl.ds(o, _CHUNK)]], R2[s], GS[s])

    def wait_gathers(s):
        for rr, ww in ((R0[s], w0), (R1[s], w1), (R2[s], w2)):
            pltpu.make_async_copy(ww.at[i0_v.at[pl.ds(0, _CHUNK)]], rr,
                                  GS[s]).wait()

    _HALF = _CHUNK // 2
    O = [oa, ob]

    def wait_write(h):
        pltpu.make_async_copy(O[h], out_hbm.at[pl.ds(out_base, _HALF)],
                              WS[h]).wait()

    def process_chunk(s, o, first):
        for h in range(2):
            if not first:
                wait_write(h)

            def add_row(row, _, s=s, h=h):
                rr = row + jnp.int32(h * _HALF)
                for cc in range(_DIM // 16):
                    sl = pl.ds(cc * 16, 16)
                    O[h][row, sl] = (R0[s][rr, sl] + R1[s][rr, sl]
                                     + R2[s][rr, sl])
                return jnp.int32(0)

            lax.fori_loop(jnp.int32(0), jnp.int32(_HALF), add_row,
                          jnp.int32(0))
            pltpu.async_copy(
                O[h], out_hbm.at[pl.ds(out_base + o + c32(h * _HALF),
                                       _HALF)], WS[h])

    npair = _NCHUNK // 2
    issue_gathers(c32(0), 0)

    def pair_step(jj, _):
        a = jj * c32(2 * _CHUNK)
        b = a + c32(_CHUNK)

        issue_gathers(b, 1)
        wait_gathers(0)

        @pl.when(jj > c32(0))
        def _():
            process_chunk(0, a, first=False)

        @pl.when(jj == c32(0))
        def _():
            process_chunk(0, a, first=True)

        @pl.when(jj < c32(npair - 1))
        def _():
            issue_gathers(a + c32(2 * _CHUNK), 0)
        wait_gathers(1)
        process_chunk(1, b, first=False)
        return jnp.int32(0)

    lax.fori_loop(jnp.int32(0), jnp.int32(npair), pair_step, jnp.int32(0))
    wait_write(0)
    wait_write(1)


@jax.jit
def kernel(tokens, W0, W1, W2):
    tok32 = tokens.astype(jnp.int32)
    padded = jnp.zeros((_BATCH, _ROWPAD), jnp.int32)
    padded = padded.at[:, 4:4 + _SEQ].set(tok32).reshape(-1)

    mesh = plsc.VectorSubcoreMesh(core_axis_name="c", subcore_axis_name="s")
    kfn = pl.kernel(
        _body,
        out_type=jax.ShapeDtypeStruct((_BATCH * _SEQ, _DIM), jnp.float32),
        mesh=mesh,
        scratch_types=[
            pltpu.VMEM((_BPW + 8,), jnp.int32),
            pltpu.VMEM((_BPW,), jnp.int32),
            pltpu.VMEM((_BPW,), jnp.int32),
            pltpu.VMEM((_BPW,), jnp.int32),
            pltpu.VMEM((_CHUNK, _DIM), jnp.float32),
            pltpu.VMEM((_CHUNK, _DIM), jnp.float32),
            pltpu.VMEM((_CHUNK, _DIM), jnp.float32),
            pltpu.VMEM((_CHUNK, _DIM), jnp.float32),
            pltpu.VMEM((_CHUNK, _DIM), jnp.float32),
            pltpu.VMEM((_CHUNK, _DIM), jnp.float32),
            pltpu.VMEM((_CHUNK // 2, _DIM), jnp.float32),
            pltpu.VMEM((_CHUNK // 2, _DIM), jnp.float32),
            pltpu.SemaphoreType.DMA,
            pltpu.SemaphoreType.DMA,
            pltpu.SemaphoreType.DMA,
            pltpu.SemaphoreType.DMA,
        ],
    )
    out = kfn(padded, W0, W1, W2)
    return out.reshape(_BATCH, _SEQ, _DIM)


# speedup vs baseline: 1.0948x; 1.0948x over previous
import jax
import jax.numpy as jnp
import numpy as np
from jax import lax
from jax.experimental import pallas as pl
from jax.experimental.pallas import tpu as pltpu
from jax.experimental.pallas import tpu_sc as plsc

_PRIME = 1000000007
_VOCAB = 32768
_DIM = 512
_BATCH = 4
_SEQ = 8192
_NW = 32
_BPW = (_BATCH * _SEQ) // _NW
_CHUNK = 32
_NCHUNK = _BPW // _CHUNK
_ROWPAD = _SEQ + 16
_WPR = _SEQ // _BPW

_POW = [pow(_PRIME, j, 2 ** 32) for j in range(5)]
_P = [np.int32(x - 2 ** 32 if x >= 2 ** 31 else x) for x in _POW]


def _body(tok_hbm, w0, w1, w2, out_hbm, tok_v, i0_v, i1_v, i2_v,
          r0a, r1a, r2a, r0b, r1b, r2b, gsa, gsb, wsa, wsb):
    c32 = jnp.int32
    wid = lax.axis_index("s") * c32(2) + lax.axis_index("c")
    brow = wid // c32(_WPR)
    coff = wid % c32(_WPR)
    tok_base = brow * c32(_ROWPAD) + coff * c32(_BPW)
    pltpu.sync_copy(tok_hbm.at[pl.ds(tok_base, _BPW + 8)], tok_v)

    def hash_step(i, _):
        o = i * jnp.int32(16)
        t0 = tok_v[pl.ds(o + 4, 16)]
        t1 = tok_v[pl.ds(o + 3, 16)]
        t2 = tok_v[pl.ds(o + 2, 16)]
        t3 = tok_v[pl.ds(o + 1, 16)]
        t4 = tok_v[pl.ds(o, 16)]
        h3 = t0 * _P[2] + t1 * _P[1] + t2
        h4 = t0 * _P[3] + t1 * _P[2] + t2 * _P[1] + t3
        h5 = t0 * _P[4] + t1 * _P[3] + t2 * _P[2] + t3 * _P[1] + t4
        m = jnp.int32(0x7FFF)
        i0_v[pl.ds(o, 16)] = jnp.bitwise_and(h3, m)
        i1_v[pl.ds(o, 16)] = jnp.bitwise_and(h4, m)
        i2_v[pl.ds(o, 16)] = jnp.bitwise_and(h5, m)
        return jnp.int32(0)

    lax.fori_loop(jnp.int32(0), jnp.int32(_CHUNK // 16), hash_step,
                  jnp.int32(0))

    out_base = wid * jnp.int32(_BPW)
    R0 = [r0a, r0b]
    R1 = [r1a, r1b]
    R2 = [r2a, r2b]
    GS = [gsa, gsb]
    WS = [wsa, wsb]

    def issue_gathers(o, s):
        pltpu.async_copy(w0.at[i0_v.at[pl.ds(o, _CHUNK)]], R0[s], GS[s])
        pltpu.async_copy(w1.at[i1_v.at[pl.ds(o, _CHUNK)]], R1[s], GS[s])
        pltpu.async_copy(w2.at[i2_v.at[pl.ds(o, _CHUNK)]], R2[s], GS[s])

    def wait_gathers(s):
        for rr, ww in ((R0[s], w0), (R1[s], w1), (R2[s], w2)):
            pltpu.make_async_copy(ww.at[i0_v.at[pl.ds(0, _CHUNK)]], rr,
                                  GS[s]).wait()

    def issue_write(o, s):
        pltpu.async_copy(R0[s], out_hbm.at[pl.ds(out_base + o, _CHUNK)],
                         WS[s])

    def wait_write(s):
        pltpu.make_async_copy(R0[s], out_hbm.at[pl.ds(out_base, _CHUNK)],
                              WS[s]).wait()

    def add_rows(s):
        def add_row(row, _):
            for cc in range(_DIM // 16):
                sl = pl.ds(cc * 16, 16)
                plsc.addupdate(R0[s].at[row, sl],
                               R1[s][row, sl] + R2[s][row, sl])
            return jnp.int32(0)
        lax.fori_loop(jnp.int32(0), jnp.int32(_CHUNK), add_row, jnp.int32(0))

    npair = _NCHUNK // 2
    issue_gathers(c32(0), 0)
    lax.fori_loop(jnp.int32(_CHUNK // 16), jnp.int32(_BPW // 16), hash_step,
                  jnp.int32(0))

    def pair_step(jj, _):
        a = jj * c32(2 * _CHUNK)
        b = a + c32(_CHUNK)

        @pl.when(jj > c32(0))
        def _():
            wait_write(1)
        issue_gathers(b, 1)
        wait_gathers(0)
        add_rows(0)
        issue_write(a, 0)
        wait_write(0)

        @pl.when(jj < c32(npair - 1))
        def _():
            issue_gathers(a + c32(2 * _CHUNK), 0)
        wait_gathers(1)
        add_rows(1)
        issue_write(b, 1)
        return jnp.int32(0)

    lax.fori_loop(jnp.int32(0), jnp.int32(npair), pair_step, jnp.int32(0))
    wait_write(1)


@jax.jit
def kernel(tokens, W0, W1, W2):
    tok32 = tokens.astype(jnp.int32)
    padded = jnp.zeros((_BATCH, _ROWPAD), jnp.int32)
    padded = padded.at[:, 4:4 + _SEQ].set(tok32).reshape(-1)

    mesh = plsc.VectorSubcoreMesh(core_axis_name="c", subcore_axis_name="s")
    kfn = pl.kernel(
        _body,
        out_type=jax.ShapeDtypeStruct((_BATCH * _SEQ, _DIM), jnp.float32),
        mesh=mesh,
        scratch_types=[
            pltpu.VMEM((_BPW + 8,), jnp.int32),
            pltpu.VMEM((_BPW,), jnp.int32),
            pltpu.VMEM((_BPW,), jnp.int32),
            pltpu.VMEM((_BPW,), jnp.int32),
            pltpu.VMEM((_CHUNK, _DIM), jnp.float32),
            pltpu.VMEM((_CHUNK, _DIM), jnp.float32),
            pltpu.VMEM((_CHUNK, _DIM), jnp.float32),
            pltpu.VMEM((_CHUNK, _DIM), jnp.float32),
            pltpu.VMEM((_CHUNK, _DIM), jnp.float32),
            pltpu.VMEM((_CHUNK, _DIM), jnp.float32),
            pltpu.SemaphoreType.DMA,
            pltpu.SemaphoreType.DMA,
            pltpu.SemaphoreType.DMA,
            pltpu.SemaphoreType.DMA,
        ],
    )
    out = kfn(padded, W0, W1, W2)
    return out.reshape(_BATCH, _SEQ, _DIM)
